# Initial kernel scaffold; baseline (speedup 1.0000x reference)
import jax
import jax.numpy as jnp
from jax.experimental import pallas as pl


def kernel(feature, edge_index, W, gamma, beta):
    raise NotImplementedError("write your pallas kernel here")


# speedup vs baseline: 2.9780x; 2.9780x over previous
import functools

import jax
import jax.numpy as jnp
from jax import lax
from jax.experimental import pallas as pl
from jax.experimental.pallas import tpu as pltpu
from jax.experimental.pallas import tpu_sc as plsc

N = 10000
E = 320000
D = 128
EPSILON = 1e-5

NTILES = 32
EPAD = 327680
EDGES_PER_TILE = EPAD // NTILES
CHUNK = 128
CHUNKS_PER_TILE = EDGES_PER_TILE // CHUNK
NROWS = 10240
ROWS_PER_SUBCORE = NROWS // 16


def _sc_body(feat_hbm, src_hbm, dst_hbm, out_hbm, src_v, dst_v, rows_v, agg_s, sem):
    c = lax.axis_index("c")
    s = lax.axis_index("s")
    wid = c * 16 + s

    zero16 = jnp.zeros((16,), jnp.float32)

    def _zrow(i, carry):
        for j in range(8):
            rows_v[i, pl.ds(j * 16, 16)] = zero16
        return carry

    lax.fori_loop(0, CHUNK, _zrow, 0)

    for k in range(ROWS_PER_SUBCORE // CHUNK):
        pltpu.sync_copy(rows_v, agg_s.at[pl.ds(s * ROWS_PER_SUBCORE + k * CHUNK, CHUNK)])
    plsc.subcore_barrier()

    def _chunk(g, carry):
        base = wid * EDGES_PER_TILE + g * CHUNK
        pltpu.sync_copy(src_hbm.at[pl.ds(base, CHUNK)], src_v)
        pltpu.sync_copy(dst_hbm.at[pl.ds(base, CHUNK)], dst_v)
        pltpu.async_copy(feat_hbm.at[src_v], rows_v, sem).wait()
        pltpu.sync_copy(rows_v, agg_s.at[dst_v], add=True)
        return carry

    lax.fori_loop(0, CHUNKS_PER_TILE, _chunk, 0)
    plsc.subcore_barrier()

    for k in range(ROWS_PER_SUBCORE // CHUNK):
        r0 = s * ROWS_PER_SUBCORE + k * CHUNK
        pltpu.sync_copy(agg_s.at[pl.ds(r0, CHUNK)], rows_v)
        pltpu.sync_copy(rows_v, out_hbm.at[c, pl.ds(r0, CHUNK)])


_sc_aggregate = functools.partial(
    pl.kernel,
    mesh=plsc.VectorSubcoreMesh(core_axis_name="c", subcore_axis_name="s"),
    out_type=jax.ShapeDtypeStruct((2, NROWS, D), jnp.float32),
    scratch_types=[
        pltpu.VMEM((CHUNK,), jnp.int32),
        pltpu.VMEM((CHUNK,), jnp.int32),
        pltpu.VMEM((CHUNK, D), jnp.float32),
        pltpu.VMEM_SHARED((NROWS, D), jnp.float32),
        pltpu.SemaphoreType.DMA,
    ],
)(_sc_body)


def _tc_body(p_ref, w_ref, g_ref, b_ref, o_ref):
    a = p_ref[0, pl.ds(0, N), :] + p_ref[1, pl.ds(0, N), :]
    agg = lax.dot_general(
        a, w_ref[...], (((1,), (1,)), ((), ())),
        preferred_element_type=jnp.float32,
        precision=lax.Precision.HIGHEST,
    )
    mean = jnp.mean(agg, axis=0, keepdims=True)
    cent = agg - mean
    var = jnp.mean(cent * cent, axis=0, keepdims=True)
    inv = lax.rsqrt(var + EPSILON)
    o_ref[...] = jnp.maximum(cent * inv * g_ref[...] + b_ref[...], 0.0)


def kernel(feature, edge_index, W, gamma, beta):
    src = edge_index[0]
    dst = edge_index[1]
    npad = EPAD - E
    src_p = jnp.concatenate([src, jnp.full((npad,), N, jnp.int32)])
    dst_p = jnp.concatenate([dst, jnp.zeros((npad,), jnp.int32)])
    feat_p = jnp.concatenate([feature, jnp.zeros((16, D), jnp.float32)], axis=0)

    partial = _sc_aggregate(feat_p, src_p, dst_p)

    out = pl.pallas_call(
        _tc_body,
        out_shape=jax.ShapeDtypeStruct((N, D), jnp.float32),
    )(partial, W, gamma.reshape(1, D), beta.reshape(1, D))
    return out


# speedup vs baseline: 4.0098x; 1.3465x over previous
import functools

import jax
import jax.numpy as jnp
from jax import lax
from jax.experimental import pallas as pl
from jax.experimental.pallas import tpu as pltpu
from jax.experimental.pallas import tpu_sc as plsc

N = 10000
E = 320000
D = 128
EPSILON = 1e-5

NTILES = 32
EPAD = 327680
CHUNK = 128
NCHUNK = EPAD // NTILES // CHUNK
NBUF = 2
NIBUF = 4
NROWS = 10240
ROWS_PER_SUBCORE = NROWS // 16


def _sc_body(feat_hbm, ei_hbm, out_hbm, ibuf, rows, agg_s, isem, gsem, ssem):
    c = lax.axis_index("c")
    s = lax.axis_index("s")
    wid = c * 16 + s

    def start_idx(g, ib):
        pltpu.async_copy(ei_hbm.at[wid, g], ibuf.at[ib], isem)

    def wait_idx(g, ib):
        pltpu.make_async_copy(ei_hbm.at[wid, g], ibuf.at[ib], isem).wait()

    def start_gather(ib, b):
        pltpu.async_copy(feat_hbm.at[ibuf.at[ib, 0]], rows.at[b], gsem)

    def wait_gather(ib, b):
        pltpu.make_async_copy(feat_hbm.at[ibuf.at[ib, 0]], rows.at[b], gsem).wait()

    def start_scatter(ib, b):
        pltpu.async_copy(rows.at[b], agg_s.at[ibuf.at[ib, 1]], ssem, add=True)

    def wait_scatter(ib, b):
        pltpu.make_async_copy(rows.at[b], agg_s.at[ibuf.at[ib, 1]], ssem).wait()

    def step(g, slot, first=False, do_next_gather=True, do_idx=True):
        b, bp = slot % NBUF, (slot + 1) % NBUF
        ib, ibn = slot % NIBUF, (slot + 1) % NIBUF
        if not first:
            wait_scatter(ib, bp)
        if do_next_gather:
            wait_idx(g + 1, ibn)
            start_gather(ibn, bp)
        wait_gather(ib, b)
        if do_idx:
            start_idx(g + 4, ib)
        start_scatter(ib, b)

    for g in range(NIBUF):
        start_idx(g, g)
    wait_idx(0, 0)
    start_gather(0, 0)

    zero16 = jnp.zeros((16,), jnp.float32)

    def _zrow(i, carry):
        for j in range(8):
            rows[1, i, pl.ds(j * 16, 16)] = zero16
        return carry

    lax.fori_loop(0, CHUNK, _zrow, 0)
    for k in range(ROWS_PER_SUBCORE // CHUNK):
        pltpu.sync_copy(rows.at[1],
                        agg_s.at[pl.ds(s * ROWS_PER_SUBCORE + k * CHUNK, CHUNK)])
    plsc.subcore_barrier()

    step(0, 0, first=True)
    for g in range(1, NIBUF):
        step(g, g)

    def _main(t, carry):
        for b in range(NIBUF):
            step(NIBUF * t + b, b)
        return carry

    lax.fori_loop(1, NCHUNK // NIBUF - 1, _main, 0)

    for g in range(NCHUNK - NIBUF, NCHUNK):
        step(g, g % NIBUF,
             do_next_gather=(g + 1 < NCHUNK), do_idx=(g + 4 < NCHUNK))
    wait_scatter((NCHUNK - 1) % NIBUF, (NCHUNK - 1) % NBUF)

    plsc.subcore_barrier()

    for k in range(ROWS_PER_SUBCORE // CHUNK):
        r0 = s * ROWS_PER_SUBCORE + k * CHUNK
        pltpu.sync_copy(agg_s.at[pl.ds(r0, CHUNK)], out_hbm.at[c, pl.ds(r0, CHUNK)])


_sc_aggregate = functools.partial(
    pl.kernel,
    mesh=plsc.VectorSubcoreMesh(core_axis_name="c", subcore_axis_name="s"),
    out_type=jax.ShapeDtypeStruct((2, NROWS, D), jnp.float32),
    scratch_types=[
        pltpu.VMEM((NIBUF, 2, CHUNK), jnp.int32),
        pltpu.VMEM((NBUF, CHUNK, D), jnp.float32),
        pltpu.VMEM_SHARED((NROWS, D), jnp.float32),
        pltpu.SemaphoreType.DMA,
        pltpu.SemaphoreType.DMA,
        pltpu.SemaphoreType.DMA,
    ],
)(_sc_body)


def _tc_body(p_ref, w_ref, g_ref, b_ref, o_ref):
    a = p_ref[0, pl.ds(0, N), :] + p_ref[1, pl.ds(0, N), :]
    agg = lax.dot_general(
        a, w_ref[...], (((1,), (1,)), ((), ())),
        preferred_element_type=jnp.float32,
        precision=lax.Precision.HIGHEST,
    )
    mean = jnp.mean(agg, axis=0, keepdims=True)
    cent = agg - mean
    var = jnp.mean(cent * cent, axis=0, keepdims=True)
    inv = lax.rsqrt(var + EPSILON)
    o_ref[...] = jnp.maximum(cent * inv * g_ref[...] + b_ref[...], 0.0)


def kernel(feature, edge_index, W, gamma, beta):
    src = edge_index[0]
    dst = edge_index[1]
    npad = EPAD - E
    src_p = jnp.concatenate([src, jnp.full((npad,), N, jnp.int32)])
    dst_p = jnp.concatenate([dst, jnp.zeros((npad,), jnp.int32)])
    feat_p = jnp.concatenate([feature, jnp.zeros((16, D), jnp.float32)], axis=0)
    ei = jnp.stack([src_p.reshape(NTILES, NCHUNK, CHUNK),
                    dst_p.reshape(NTILES, NCHUNK, CHUNK)], axis=2)

    partial = _sc_aggregate(feat_p, ei)

    out = pl.pallas_call(
        _tc_body,
        out_shape=jax.ShapeDtypeStruct((N, D), jnp.float32),
    )(partial, W, gamma.reshape(1, D), beta.reshape(1, D))
    return out


# speedup vs baseline: 5.4579x; 1.3611x over previous
import functools

import jax
import jax.numpy as jnp
from jax import lax
from jax.experimental import pallas as pl
from jax.experimental.pallas import tpu as pltpu
from jax.experimental.pallas import tpu_sc as plsc

N = 10000
E = 320000
D = 128
DH = D // 2
EPSILON = 1e-5

EPAD = 327680
CHUNK = 128
NCHUNK = EPAD // 16 // CHUNK
NB = 4
NI = 8
GA = 2
NROWS = 10240
RPS = NROWS // 16


def _sc_body(feat_hbm, ei_hbm, out_hbm, ibuf, rows, agg_s, isem, gsem, ssem):
    c = lax.axis_index("c")
    s = lax.axis_index("s")

    def start_idx(g, ib):
        pltpu.async_copy(ei_hbm.at[c, s, g], ibuf.at[ib], isem)

    def wait_idx(g, ib):
        pltpu.make_async_copy(ei_hbm.at[c, s, g], ibuf.at[ib], isem).wait()

    def start_gather(ib, b):
        pltpu.async_copy(feat_hbm.at[ibuf.at[ib, 0]], rows.at[b], gsem)

    def wait_gather(ib, b):
        pltpu.make_async_copy(feat_hbm.at[ibuf.at[ib, 0]], rows.at[b], gsem).wait()

    def start_scatter(ib, b):
        pltpu.async_copy(rows.at[b], agg_s.at[ibuf.at[ib, 1]], ssem, add=True)

    def wait_scatter(ib, b):
        pltpu.make_async_copy(rows.at[b], agg_s.at[ibuf.at[ib, 1]], ssem).wait()

    for g in range(NI - GA):
        start_idx(g, g)

    zero16 = jnp.zeros((16,), jnp.float32)

    def _zrow(i, carry):
        for j in range(DH // 16):
            rows[NB - 1, i, pl.ds(j * 16, 16)] = zero16
        return carry

    lax.fori_loop(0, CHUNK, _zrow, 0)
    for k in range(RPS // CHUNK):
        pltpu.sync_copy(rows.at[NB - 1],
                        agg_s.at[pl.ds(s * RPS + k * CHUNK, CHUNK)])
    plsc.subcore_barrier()

    def step(g, slot, first=False, do_idx=True, do_gather=True):
        if not first:
            wait_scatter((slot - GA) % NI, (slot - GA) % NB)
        if do_idx:
            start_idx(g + NI - GA, (slot - GA) % NI)
        if do_gather:
            wait_idx(g + GA, (slot + GA) % NI)
            start_gather((slot + GA) % NI, (slot + GA) % NB)
        wait_gather(slot % NI, slot % NB)
        start_scatter(slot % NI, slot % NB)

    for g in range(GA):
        wait_idx(g, g)
        start_gather(g, g)

    for g in range(NI):
        step(g, g, first=(g < GA))

    def _main(t, carry):
        for b in range(NI):
            step(NI * t + b, b)
        return carry

    lax.fori_loop(1, NCHUNK // NI - 1, _main, 0)

    for g in range(NCHUNK - NI, NCHUNK):
        step(g, g % NI,
             do_idx=(g + NI - GA < NCHUNK), do_gather=(g + GA < NCHUNK))
    for g in range(NCHUNK - GA, NCHUNK):
        wait_scatter(g % NI, g % NB)

    plsc.subcore_barrier()

    for k in range(RPS // CHUNK):
        r0 = s * RPS + k * CHUNK
        pltpu.sync_copy(agg_s.at[pl.ds(r0, CHUNK)], out_hbm.at[c, pl.ds(r0, CHUNK)])


_sc_aggregate = functools.partial(
    pl.kernel,
    mesh=plsc.VectorSubcoreMesh(core_axis_name="c", subcore_axis_name="s"),
    compiler_params=pltpu.CompilerParams(use_tc_tiling_on_sc=False),
    out_type=jax.ShapeDtypeStruct((2, NROWS, DH), jnp.float32),
    scratch_types=[
        pltpu.VMEM((NI, 2, CHUNK), jnp.int32),
        pltpu.VMEM((NB, CHUNK, DH), jnp.float32),
        pltpu.VMEM_SHARED((NROWS, DH), jnp.float32),
        pltpu.SemaphoreType.DMA,
        pltpu.SemaphoreType.DMA,
        pltpu.SemaphoreType.DMA,
    ],
)(_sc_body)


def _tc_body(p_ref, w_ref, g_ref, b_ref, o_ref):
    a = jnp.concatenate(
        [p_ref[0, pl.ds(0, N), :], p_ref[1, pl.ds(0, N), :]], axis=1)
    agg = lax.dot_general(
        a, w_ref[...], (((1,), (1,)), ((), ())),
        preferred_element_type=jnp.float32,
        precision=lax.Precision.HIGHEST,
    )
    mean = jnp.mean(agg, axis=0, keepdims=True)
    cent = agg - mean
    var = jnp.mean(cent * cent, axis=0, keepdims=True)
    inv = lax.rsqrt(var + EPSILON)
    o_ref[...] = jnp.maximum(cent * inv * g_ref[...] + b_ref[...], 0.0)


def kernel(feature, edge_index, W, gamma, beta):
    src = edge_index[0]
    dst = edge_index[1]
    npad = EPAD - E
    src_p = jnp.concatenate([src, jnp.full((npad,), N, jnp.int32)])
    dst_p = jnp.concatenate([dst, jnp.zeros((npad,), jnp.int32)])
    feat_t = jnp.zeros((2, NROWS, DH), jnp.float32)
    feat_t = feat_t.at[:, :N, :].set(
        feature.reshape(N, 2, DH).transpose(1, 0, 2))
    feat_t = feat_t.reshape(2 * NROWS, DH)
    src_r = src_p.reshape(16, NCHUNK, CHUNK)
    dst_r = dst_p.reshape(16, NCHUNK, CHUNK)
    ei = jnp.stack([jnp.stack([src_r + c * NROWS, dst_r], axis=2)
                    for c in range(2)], axis=0)

    partial = _sc_aggregate(feat_t, ei)

    out = pl.pallas_call(
        _tc_body,
        out_shape=jax.ShapeDtypeStruct((N, D), jnp.float32),
    )(partial, W, gamma.reshape(1, D), beta.reshape(1, D))
    return out
